# baseline (speedup 1.0000x reference)
import functools

import jax
import jax.numpy as jnp
from jax import lax
from jax.experimental import pallas as pl
from jax.experimental.pallas import tpu as pltpu
from jax.experimental.pallas import tpu_sc as plsc

N_SAMPLE = 128
N_POS = 32
N_NEG = 96
POS_TH = 0.5
NEG_HI = 0.5
NROI = 20000
LANES = 128
ROWS = 160
NPAD = ROWS * LANES
G = 32
B = 4
BOTTOM = float(-3e38)


def _select_body(coords_ref, gt_ref, idx_ref, val_ref):
    rx1 = coords_ref[0, 0]
    ry1 = coords_ref[0, 1]
    rx2 = coords_ref[0, 2]
    ry2 = coords_ref[0, 3]
    rowi = lax.broadcasted_iota(jnp.int32, (ROWS, LANES), 0)
    coli = lax.broadcasted_iota(jnp.int32, (ROWS, LANES), 1)
    lin = rowi * LANES + coli
    valid = lin < NROI
    area_r = (ry2 - ry1) * (rx2 - rx1)
    max_iou = jnp.zeros((ROWS, LANES), jnp.float32)
    for g in range(G):
        xb1 = gt_ref[0, g, 0]
        yb1 = gt_ref[0, g, 1]
        xb2 = gt_ref[0, g, 2]
        yb2 = gt_ref[0, g, 3]
        ix1 = jnp.maximum(rx1, xb1)
        iy1 = jnp.maximum(ry1, yb1)
        ix2 = jnp.minimum(rx2, xb2)
        iy2 = jnp.minimum(ry2, yb2)
        inter = (iy2 - iy1) * (ix2 - ix1)
        gt_area = (yb2 - yb1) * (xb2 - xb1)
        union = area_r + gt_area - inter
        ok = (ix1 < ix2) & (iy1 < iy2)
        iou = jnp.where(ok, inter / union, 0.0)
        max_iou = jnp.maximum(max_iou, iou)

    pos_s = jnp.where(valid, jnp.where(max_iou >= POS_TH, max_iou, -1.0), BOTTOM)
    neg_s = jnp.where(valid & (max_iou < NEG_HI), -max_iou, BOTTOM)

    lane = lax.broadcasted_iota(jnp.int32, (1, LANES), 1)
    neg_inf = jnp.float32(-jnp.inf)

    def extract(s):
        m = jnp.max(s)
        cand = jnp.where(s == m, lin, jnp.int32(2**30))
        idx = jnp.min(cand)
        return jnp.where(lin == idx, neg_inf, s), idx, m

    def pos_body(it, c):
        s, ai, av = c
        s, idx, m = extract(s)
        ai = jnp.where(lane == it, idx, ai)
        av = jnp.where(lane == it, m, av)
        return s, ai, av

    def neg_body(it, c):
        s, ai, av = c
        s, idx, m = extract(s)
        ai = jnp.where(lane == it + N_POS, idx, ai)
        av = jnp.where(lane == it + N_POS, m, av)
        return s, ai, av

    acc_i = jnp.zeros((1, LANES), jnp.int32)
    acc_v = jnp.zeros((1, LANES), jnp.float32)
    _, acc_i, acc_v = lax.fori_loop(0, N_POS, pos_body, (pos_s, acc_i, acc_v))
    _, acc_i, acc_v = lax.fori_loop(0, N_NEG, neg_body, (neg_s, acc_i, acc_v))
    idx_ref[0] = acc_i
    val_ref[0] = acc_v


def _select(coords, gt_boxes):
    return pl.pallas_call(
        _select_body,
        grid=(B,),
        in_specs=[
            pl.BlockSpec((1, 4, ROWS, LANES), lambda i: (i, 0, 0, 0)),
            pl.BlockSpec((1, G, 4), lambda i: (i, 0, 0), memory_space=pltpu.SMEM),
        ],
        out_specs=[
            pl.BlockSpec((1, 1, LANES), lambda i: (i, 0, 0)),
            pl.BlockSpec((1, 1, LANES), lambda i: (i, 0, 0)),
        ],
        out_shape=[
            jax.ShapeDtypeStruct((B, 1, LANES), jnp.int32),
            jax.ShapeDtypeStruct((B, 1, LANES), jnp.float32),
        ],
    )(coords, gt_boxes)


def _ln(x):
    u = lax.bitcast_convert_type(x, jnp.int32)
    e = lax.shift_right_arithmetic(u, 23) - 127
    m = lax.bitcast_convert_type(
        (u & jnp.int32(0x7FFFFF)) | jnp.int32(0x3F800000), jnp.float32)
    big = m > jnp.float32(1.4142135623730951)
    m = jnp.where(big, m * 0.5, m)
    e = e + jnp.where(big, 1, 0)
    t = (m - 1.0) / (m + 1.0)
    t2 = t * t
    p = 1.0 + t2 * (jnp.float32(1 / 3) + t2 * (jnp.float32(1 / 5)
        + t2 * (jnp.float32(1 / 7) + t2 * jnp.float32(1 / 9))))
    return e.astype(jnp.float32) * jnp.float32(0.6931471805599453) + 2.0 * t * p


def _sc_body(roi_hbm, gt_hbm, lab_hbm, kidx_hbm, kval_hbm,
             oroi_hbm, olab_hbm, obox_hbm,
             kidx_v, kval_v, gidx_v, rows_v, gtv, labv,
             oroi_v, olab_v, obox_v, sem):
    wid = lax.axis_index("s") * 2 + lax.axis_index("c")
    img = wid // 8
    base = wid * 16
    iota = lax.iota(jnp.int32, 16)

    pltpu.sync_copy(kidx_hbm.at[pl.ds(base, 16)], kidx_v)
    pltpu.sync_copy(kval_hbm.at[pl.ds(base, 16)], kval_v)
    pltpu.sync_copy(gt_hbm.at[pl.ds(img * 4, 4)], gtv)
    pltpu.sync_copy(lab_hbm.at[pl.ds(img * G, G)], labv)

    gidx_v[...] = kidx_v[...] + img * NROI
    pltpu.async_copy(roi_hbm.at[gidx_v], rows_v, sem).wait()

    def col(c):
        return plsc.load_gather(rows_v, [iota, jnp.full((16,), c, jnp.int32)])

    x1, y1, x2, y2 = col(0), col(1), col(2), col(3)

    gplanes = [[gtv[c, pl.ds(0, 16)], gtv[c, pl.ds(16, 16)]] for c in range(4)]

    assigned = jnp.zeros((16,), jnp.int32)
    best = jnp.full((16,), -1.0, jnp.float32)
    area_r = (y2 - y1) * (x2 - x1)
    for g in range(G):
        xb1 = gplanes[0][g // 16][g % 16]
        yb1 = gplanes[1][g // 16][g % 16]
        xb2 = gplanes[2][g // 16][g % 16]
        yb2 = gplanes[3][g // 16][g % 16]
        ix1 = jnp.maximum(x1, xb1)
        iy1 = jnp.maximum(y1, yb1)
        ix2 = jnp.minimum(x2, xb2)
        iy2 = jnp.minimum(y2, yb2)
        inter = (iy2 - iy1) * (ix2 - ix1)
        gt_area = (yb2 - yb1) * (xb2 - xb1)
        union = area_r + gt_area - inter
        ok = (ix1 < ix2) & (iy1 < iy2)
        iou = jnp.where(ok, inter / union, 0.0)
        upd = iou > best
        assigned = jnp.where(upd, g, assigned)
        best = jnp.where(upd, iou, best)

    def gcol(c):
        return plsc.load_gather(gtv, [jnp.full((16,), c, jnp.int32), assigned])

    bx1, by1, bx2, by2 = gcol(0), gcol(1), gcol(2), gcol(3)

    slot = (wid % 8) * 16 + iota
    pos_valid = (slot < N_POS) & (kval_v[...] >= POS_TH)
    label = plsc.load_gather(labv, [assigned])
    label = jnp.where(pos_valid, label, 0)

    eps = jnp.float32(1.1920929e-07)
    w = x2 - x1
    h = y2 - y1
    cx = x1 + 0.5 * w
    cy = y1 + 0.5 * h
    bw = bx2 - bx1
    bh = by2 - by1
    bcx = bx1 + 0.5 * bw
    bcy = by1 + 0.5 * bh
    wc = jnp.maximum(w, eps)
    hc = jnp.maximum(h, eps)
    dx = (bcx - cx) / wc
    dy = (bcy - cy) / hc
    dw = _ln(bw / wc)
    dh = _ln(bh / hc)

    def scat(ref, c, v):
        plsc.store_scatter(ref, [iota, jnp.full((16,), c, jnp.int32)], v)

    scat(oroi_v, 0, x1)
    scat(oroi_v, 1, y1)
    scat(oroi_v, 2, x2)
    scat(oroi_v, 3, y2)
    scat(obox_v, 0, dx)
    scat(obox_v, 1, dy)
    scat(obox_v, 2, dw)
    scat(obox_v, 3, dh)
    olab_v[...] = label

    pltpu.sync_copy(oroi_v, oroi_hbm.at[pl.ds(base, 16)])
    pltpu.sync_copy(olab_v, olab_hbm.at[pl.ds(base, 16)])
    pltpu.sync_copy(obox_v, obox_hbm.at[pl.ds(base, 16)])


@functools.cache
def _sc_gather():
  return pl.kernel(
    _sc_body,
    out_type=[
        jax.ShapeDtypeStruct((B * N_SAMPLE, 4), jnp.float32),
        jax.ShapeDtypeStruct((B * N_SAMPLE,), jnp.int32),
        jax.ShapeDtypeStruct((B * N_SAMPLE, 4), jnp.float32),
    ],
    mesh=plsc.VectorSubcoreMesh(
        core_axis_name="c", subcore_axis_name="s",
        num_cores=2, num_subcores=16),
    scratch_types=[
        pltpu.VMEM((16,), jnp.int32),
        pltpu.VMEM((16,), jnp.float32),
        pltpu.VMEM((16,), jnp.int32),
        pltpu.VMEM((16, 16), jnp.float32),
        pltpu.VMEM((4, G), jnp.float32),
        pltpu.VMEM((G,), jnp.int32),
        pltpu.VMEM((16, 4), jnp.float32),
        pltpu.VMEM((16,), jnp.int32),
        pltpu.VMEM((16, 4), jnp.float32),
        pltpu.SemaphoreType.DMA,
    ],
    compiler_params=pltpu.CompilerParams(
        needs_layout_passes=False, use_tc_tiling_on_sc=False),
  )


def kernel(roi, gt_boxes, labels, image):
    del image
    coords = jnp.transpose(roi, (0, 2, 1))
    coords = jnp.pad(coords, ((0, 0), (0, 0), (0, NPAD - NROI)))
    coords = coords.reshape(B, 4, ROWS, LANES)
    keep_idx, keep_val = _select(coords, gt_boxes)

    roi_pad = jnp.pad(roi.reshape(B * NROI, 4), ((0, 0), (0, 12)))
    gt_planes = jnp.transpose(gt_boxes, (0, 2, 1)).reshape(B * 4, G)
    lab_flat = labels.reshape(B * G).astype(jnp.int32)
    sroi, slab, sbox = _sc_gather()(
        roi_pad, gt_planes, lab_flat,
        keep_idx.reshape(B * N_SAMPLE), keep_val.reshape(B * N_SAMPLE))
    return (sroi.reshape(B, N_SAMPLE, 4),
            slab.reshape(B, N_SAMPLE),
            sbox.reshape(B, N_SAMPLE, 4))


# speedup vs baseline: 1.3057x; 1.3057x over previous
import functools

import jax
import jax.numpy as jnp
from jax import lax
from jax.experimental import pallas as pl
from jax.experimental.pallas import tpu as pltpu
from jax.experimental.pallas import tpu_sc as plsc

N_SAMPLE = 128
N_POS = 32
N_NEG = 96
POS_TH = 0.5
NEG_HI = 0.5
NROI = 20000
LANES = 128
ROWS = 160
NPAD = ROWS * LANES
G = 32
B = 4
BOTTOM = float(-3e38)


def _lane_reduce(v, op):
    for sh in (1, 2, 4, 8, 16, 32, 64):
        v = op(v, pltpu.roll(v, sh, 1))
    return v


def _select_body(coords_ref, gt_ref, idx_ref, val_ref):
    rowi = lax.broadcasted_iota(jnp.int32, (ROWS, LANES), 0)
    coli = lax.broadcasted_iota(jnp.int32, (ROWS, LANES), 1)
    lin = rowi * LANES + coli
    valid = lin < NROI
    lane = lax.broadcasted_iota(jnp.int32, (1, LANES), 1)
    neg_inf = jnp.float32(-jnp.inf)

    pos_list, neg_list = [], []
    for b in range(B):
        rx1 = coords_ref[b, 0]
        ry1 = coords_ref[b, 1]
        rx2 = coords_ref[b, 2]
        ry2 = coords_ref[b, 3]
        area_r = (ry2 - ry1) * (rx2 - rx1)
        max_iou = jnp.zeros((ROWS, LANES), jnp.float32)
        for g in range(G):
            xb1 = gt_ref[b, g, 0]
            yb1 = gt_ref[b, g, 1]
            xb2 = gt_ref[b, g, 2]
            yb2 = gt_ref[b, g, 3]
            ix1 = jnp.maximum(rx1, xb1)
            iy1 = jnp.maximum(ry1, yb1)
            ix2 = jnp.minimum(rx2, xb2)
            iy2 = jnp.minimum(ry2, yb2)
            inter = (iy2 - iy1) * (ix2 - ix1)
            gt_area = (yb2 - yb1) * (xb2 - xb1)
            union = area_r + gt_area - inter
            ok = (ix1 < ix2) & (iy1 < iy2)
            iou = jnp.where(ok, inter / union, 0.0)
            max_iou = jnp.maximum(max_iou, iou)
        pos_list.append(
            jnp.where(valid, jnp.where(max_iou >= POS_TH, max_iou, -1.0), BOTTOM))
        neg_list.append(jnp.where(valid & (max_iou < NEG_HI), -max_iou, BOTTOM))

    def extract(s):
        m = _lane_reduce(jnp.max(s, axis=0, keepdims=True), jnp.maximum)
        cand = jnp.where(s == m, lin, jnp.int32(2**30))
        idx = _lane_reduce(jnp.min(cand, axis=0, keepdims=True), jnp.minimum)
        return jnp.where(lin == idx, neg_inf, s), idx, m

    def body(it, c, do_pos):
        pos, neg, ai, av = c
        pos, neg, ai, av = list(pos), list(neg), list(ai), list(av)
        for b in range(B):
            if do_pos:
                pos[b], idx, m = extract(pos[b])
                ai[b] = jnp.where(lane == it, idx, ai[b])
                av[b] = jnp.where(lane == it, m, av[b])
            neg[b], idx, m = extract(neg[b])
            ai[b] = jnp.where(lane == it + N_POS, idx, ai[b])
            av[b] = jnp.where(lane == it + N_POS, m, av[b])
        return tuple(pos), tuple(neg), tuple(ai), tuple(av)

    acc_i = tuple(jnp.zeros((1, LANES), jnp.int32) for _ in range(B))
    acc_v = tuple(jnp.zeros((1, LANES), jnp.float32) for _ in range(B))
    carry = (tuple(pos_list), tuple(neg_list), acc_i, acc_v)
    carry = lax.fori_loop(0, N_POS, lambda it, c: body(it, c, True), carry)
    carry = lax.fori_loop(
        N_POS, N_NEG, lambda it, c: body(it, c, False), carry)
    _, _, acc_i, acc_v = carry
    for b in range(B):
        idx_ref[b] = acc_i[b]
        val_ref[b] = acc_v[b]


def _select(coords, gt_boxes):
    return pl.pallas_call(
        _select_body,
        in_specs=[
            pl.BlockSpec(memory_space=pltpu.VMEM),
            pl.BlockSpec(memory_space=pltpu.SMEM),
        ],
        out_shape=[
            jax.ShapeDtypeStruct((B, 1, LANES), jnp.int32),
            jax.ShapeDtypeStruct((B, 1, LANES), jnp.float32),
        ],
    )(coords, gt_boxes)


def _ln(x):
    u = lax.bitcast_convert_type(x, jnp.int32)
    e = lax.shift_right_arithmetic(u, 23) - 127
    m = lax.bitcast_convert_type(
        (u & jnp.int32(0x7FFFFF)) | jnp.int32(0x3F800000), jnp.float32)
    big = m > jnp.float32(1.4142135623730951)
    m = jnp.where(big, m * 0.5, m)
    e = e + jnp.where(big, 1, 0)
    t = (m - 1.0) / (m + 1.0)
    t2 = t * t
    p = 1.0 + t2 * (jnp.float32(1 / 3) + t2 * (jnp.float32(1 / 5)
        + t2 * (jnp.float32(1 / 7) + t2 * jnp.float32(1 / 9))))
    return e.astype(jnp.float32) * jnp.float32(0.6931471805599453) + 2.0 * t * p


def _sc_body(roi_hbm, gt_hbm, lab_hbm, kidx_hbm, kval_hbm,
             oroi_hbm, olab_hbm, obox_hbm,
             kidx_v, kval_v, gidx_v, rows_v, gtv, labv,
             oroi_v, olab_v, obox_v, sem):
    wid = lax.axis_index("s") * 2 + lax.axis_index("c")
    img = wid // 8
    base = wid * 16
    iota = lax.iota(jnp.int32, 16)

    pltpu.sync_copy(kidx_hbm.at[pl.ds(base, 16)], kidx_v)
    pltpu.sync_copy(kval_hbm.at[pl.ds(base, 16)], kval_v)
    pltpu.sync_copy(gt_hbm.at[pl.ds(img * 4, 4)], gtv)
    pltpu.sync_copy(lab_hbm.at[pl.ds(img * G, G)], labv)

    gidx_v[...] = kidx_v[...] + img * NROI
    pltpu.async_copy(roi_hbm.at[gidx_v], rows_v, sem).wait()

    def col(c):
        return plsc.load_gather(rows_v, [iota, jnp.full((16,), c, jnp.int32)])

    x1, y1, x2, y2 = col(0), col(1), col(2), col(3)

    gplanes = [[gtv[c, pl.ds(0, 16)], gtv[c, pl.ds(16, 16)]] for c in range(4)]

    assigned = jnp.zeros((16,), jnp.int32)
    best = jnp.full((16,), -1.0, jnp.float32)
    area_r = (y2 - y1) * (x2 - x1)
    for g in range(G):
        xb1 = gplanes[0][g // 16][g % 16]
        yb1 = gplanes[1][g // 16][g % 16]
        xb2 = gplanes[2][g // 16][g % 16]
        yb2 = gplanes[3][g // 16][g % 16]
        ix1 = jnp.maximum(x1, xb1)
        iy1 = jnp.maximum(y1, yb1)
        ix2 = jnp.minimum(x2, xb2)
        iy2 = jnp.minimum(y2, yb2)
        inter = (iy2 - iy1) * (ix2 - ix1)
        gt_area = (yb2 - yb1) * (xb2 - xb1)
        union = area_r + gt_area - inter
        ok = (ix1 < ix2) & (iy1 < iy2)
        iou = jnp.where(ok, inter / union, 0.0)
        upd = iou > best
        assigned = jnp.where(upd, g, assigned)
        best = jnp.where(upd, iou, best)

    def gcol(c):
        return plsc.load_gather(gtv, [jnp.full((16,), c, jnp.int32), assigned])

    bx1, by1, bx2, by2 = gcol(0), gcol(1), gcol(2), gcol(3)

    slot = (wid % 8) * 16 + iota
    pos_valid = (slot < N_POS) & (kval_v[...] >= POS_TH)
    label = plsc.load_gather(labv, [assigned])
    label = jnp.where(pos_valid, label, 0)

    eps = jnp.float32(1.1920929e-07)
    w = x2 - x1
    h = y2 - y1
    cx = x1 + 0.5 * w
    cy = y1 + 0.5 * h
    bw = bx2 - bx1
    bh = by2 - by1
    bcx = bx1 + 0.5 * bw
    bcy = by1 + 0.5 * bh
    wc = jnp.maximum(w, eps)
    hc = jnp.maximum(h, eps)
    dx = (bcx - cx) / wc
    dy = (bcy - cy) / hc
    dw = _ln(bw / wc)
    dh = _ln(bh / hc)

    def scat(ref, c, v):
        plsc.store_scatter(ref, [iota, jnp.full((16,), c, jnp.int32)], v)

    scat(oroi_v, 0, x1)
    scat(oroi_v, 1, y1)
    scat(oroi_v, 2, x2)
    scat(oroi_v, 3, y2)
    scat(obox_v, 0, dx)
    scat(obox_v, 1, dy)
    scat(obox_v, 2, dw)
    scat(obox_v, 3, dh)
    olab_v[...] = label

    pltpu.sync_copy(oroi_v, oroi_hbm.at[pl.ds(base, 16)])
    pltpu.sync_copy(olab_v, olab_hbm.at[pl.ds(base, 16)])
    pltpu.sync_copy(obox_v, obox_hbm.at[pl.ds(base, 16)])


@functools.cache
def _sc_gather():
  return pl.kernel(
    _sc_body,
    out_type=[
        jax.ShapeDtypeStruct((B * N_SAMPLE, 4), jnp.float32),
        jax.ShapeDtypeStruct((B * N_SAMPLE,), jnp.int32),
        jax.ShapeDtypeStruct((B * N_SAMPLE, 4), jnp.float32),
    ],
    mesh=plsc.VectorSubcoreMesh(
        core_axis_name="c", subcore_axis_name="s",
        num_cores=2, num_subcores=16),
    scratch_types=[
        pltpu.VMEM((16,), jnp.int32),
        pltpu.VMEM((16,), jnp.float32),
        pltpu.VMEM((16,), jnp.int32),
        pltpu.VMEM((16, 16), jnp.float32),
        pltpu.VMEM((4, G), jnp.float32),
        pltpu.VMEM((G,), jnp.int32),
        pltpu.VMEM((16, 4), jnp.float32),
        pltpu.VMEM((16,), jnp.int32),
        pltpu.VMEM((16, 4), jnp.float32),
        pltpu.SemaphoreType.DMA,
    ],
    compiler_params=pltpu.CompilerParams(
        needs_layout_passes=False, use_tc_tiling_on_sc=False),
  )


def kernel(roi, gt_boxes, labels, image):
    del image
    coords = jnp.transpose(roi, (0, 2, 1))
    coords = jnp.pad(coords, ((0, 0), (0, 0), (0, NPAD - NROI)))
    coords = coords.reshape(B, 4, ROWS, LANES)
    keep_idx, keep_val = _select(coords, gt_boxes)

    roi_pad = jnp.pad(roi.reshape(B * NROI, 4), ((0, 0), (0, 12)))
    gt_planes = jnp.transpose(gt_boxes, (0, 2, 1)).reshape(B * 4, G)
    lab_flat = labels.reshape(B * G).astype(jnp.int32)
    sroi, slab, sbox = _sc_gather()(
        roi_pad, gt_planes, lab_flat,
        keep_idx.reshape(B * N_SAMPLE), keep_val.reshape(B * N_SAMPLE))
    return (sroi.reshape(B, N_SAMPLE, 4),
            slab.reshape(B, N_SAMPLE),
            sbox.reshape(B, N_SAMPLE, 4))


# speedup vs baseline: 1.6204x; 1.2411x over previous
import functools

import jax
import jax.numpy as jnp
from jax import lax
from jax.experimental import pallas as pl
from jax.experimental.pallas import tpu as pltpu
from jax.experimental.pallas import tpu_sc as plsc

N_SAMPLE = 128
N_POS = 32
N_NEG = 96
POS_TH = 0.5
NEG_HI = 0.5
NROI = 20000
LANES = 128
ROWS = 160
NPAD = ROWS * LANES
G = 32
B = 4
BOTTOM = float(-3e38)


NCH = ROWS // 8


def _comb(a, b):
    v1, l1 = a
    v2, l2 = b
    take2 = (v2 > v1) | ((v2 == v1) & (l2 < l1))
    return jnp.where(take2, v2, v1), jnp.where(take2, l2, l1)


def _lin_chunk(i):
    r = lax.broadcasted_iota(jnp.int32, (8, LANES), 0)
    c = lax.broadcasted_iota(jnp.int32, (8, LANES), 1)
    return (r + i * 8) * LANES + c


def _extract(chunks, lins):
    pairs = list(zip(chunks, lins))
    while len(pairs) > 1:
        nxt = [_comb(pairs[j], pairs[j + 1]) for j in range(0, len(pairs) - 1, 2)]
        if len(pairs) % 2:
            nxt.append(pairs[-1])
        pairs = nxt
    v, l = pairs[0]
    for half in (4, 2, 1):
        v, l = _comb((v[:half], l[:half]), (v[half:2 * half], l[half:2 * half]))
    for sh in (1, 2, 4, 8, 16, 32, 64):
        v, l = _comb((v, l), (pltpu.roll(v, sh, 1), pltpu.roll(l, sh, 1)))
    neg_inf = jnp.float32(-jnp.inf)
    new_chunks = [jnp.where(lins[i] == l, neg_inf, chunks[i])
                  for i in range(len(chunks))]
    return new_chunks, l, v


def _select_body(coords_ref, gt_ref, idx_ref, val_ref):
    lane = lax.broadcasted_iota(jnp.int32, (1, LANES), 1)
    lins = [_lin_chunk(i) for i in range(NCH)]

    pos_list = [[None] * NCH for _ in range(B)]
    neg_list = [[None] * NCH for _ in range(B)]
    for b in range(B):
        gts = [[gt_ref[b, g, c] for c in range(4)] for g in range(G)]
        for i in range(NCH):
            rx1 = coords_ref[b, 0, pl.ds(i * 8, 8), :]
            ry1 = coords_ref[b, 1, pl.ds(i * 8, 8), :]
            rx2 = coords_ref[b, 2, pl.ds(i * 8, 8), :]
            ry2 = coords_ref[b, 3, pl.ds(i * 8, 8), :]
            area_r = (ry2 - ry1) * (rx2 - rx1)
            max_iou = jnp.zeros((8, LANES), jnp.float32)
            for g in range(G):
                xb1, yb1, xb2, yb2 = gts[g]
                ix1 = jnp.maximum(rx1, xb1)
                iy1 = jnp.maximum(ry1, yb1)
                ix2 = jnp.minimum(rx2, xb2)
                iy2 = jnp.minimum(ry2, yb2)
                inter = (iy2 - iy1) * (ix2 - ix1)
                gt_area = (yb2 - yb1) * (xb2 - xb1)
                union = area_r + gt_area - inter
                ok = (ix1 < ix2) & (iy1 < iy2)
                iou = jnp.where(ok, inter / union, 0.0)
                max_iou = jnp.maximum(max_iou, iou)
            p = jnp.where(max_iou >= POS_TH, max_iou, -1.0)
            n = jnp.where(max_iou < NEG_HI, -max_iou, BOTTOM)
            if i == NCH - 1:
                pad = lins[i] >= NROI
                p = jnp.where(pad, BOTTOM, p)
                n = jnp.where(pad, BOTTOM, n)
            pos_list[b][i] = p
            neg_list[b][i] = n

    def body(it, c, do_pos):
        pos = [list(t) for t in c[0]]
        neg = [list(t) for t in c[1]]
        ai = list(c[2])
        av = list(c[3])
        for b in range(B):
            if do_pos:
                pos[b], l, v = _extract(pos[b], lins)
                ai[b] = jnp.where(lane == it, l, ai[b])
                av[b] = jnp.where(lane == it, v, av[b])
            neg[b], l, v = _extract(neg[b], lins)
            ai[b] = jnp.where(lane == it + N_POS, l, ai[b])
            av[b] = jnp.where(lane == it + N_POS, v, av[b])
        return (tuple(map(tuple, pos)), tuple(map(tuple, neg)),
                tuple(ai), tuple(av))

    acc_i = tuple(jnp.zeros((1, LANES), jnp.int32) for _ in range(B))
    acc_v = tuple(jnp.zeros((1, LANES), jnp.float32) for _ in range(B))
    carry = (tuple(map(tuple, pos_list)), tuple(map(tuple, neg_list)),
             acc_i, acc_v)
    carry = lax.fori_loop(0, N_POS, lambda it, c: body(it, c, True), carry)
    carry = lax.fori_loop(N_POS, N_NEG, lambda it, c: body(it, c, False), carry)
    _, _, acc_i, acc_v = carry
    for b in range(B):
        idx_ref[b] = acc_i[b]
        val_ref[b] = acc_v[b]


def _select(coords, gt_boxes):
    return pl.pallas_call(
        _select_body,
        in_specs=[
            pl.BlockSpec(memory_space=pltpu.VMEM),
            pl.BlockSpec(memory_space=pltpu.SMEM),
        ],
        out_shape=[
            jax.ShapeDtypeStruct((B, 1, LANES), jnp.int32),
            jax.ShapeDtypeStruct((B, 1, LANES), jnp.float32),
        ],
    )(coords, gt_boxes)


def _ln(x):
    u = lax.bitcast_convert_type(x, jnp.int32)
    e = lax.shift_right_arithmetic(u, 23) - 127
    m = lax.bitcast_convert_type(
        (u & jnp.int32(0x7FFFFF)) | jnp.int32(0x3F800000), jnp.float32)
    big = m > jnp.float32(1.4142135623730951)
    m = jnp.where(big, m * 0.5, m)
    e = e + jnp.where(big, 1, 0)
    t = (m - 1.0) / (m + 1.0)
    t2 = t * t
    p = 1.0 + t2 * (jnp.float32(1 / 3) + t2 * (jnp.float32(1 / 5)
        + t2 * (jnp.float32(1 / 7) + t2 * jnp.float32(1 / 9))))
    return e.astype(jnp.float32) * jnp.float32(0.6931471805599453) + 2.0 * t * p


def _sc_body(roi_hbm, gt_hbm, lab_hbm, kidx_hbm, kval_hbm,
             oroi_hbm, olab_hbm, obox_hbm,
             kidx_v, kval_v, gidx_v, rows_v, gtv, labv,
             oroi_v, olab_v, obox_v, sem):
    wid = lax.axis_index("s") * 2 + lax.axis_index("c")
    img = wid // 8
    base = wid * 16
    iota = lax.iota(jnp.int32, 16)

    pltpu.sync_copy(kidx_hbm.at[pl.ds(base, 16)], kidx_v)
    pltpu.sync_copy(kval_hbm.at[pl.ds(base, 16)], kval_v)
    pltpu.sync_copy(gt_hbm.at[pl.ds(img * 4, 4)], gtv)
    pltpu.sync_copy(lab_hbm.at[pl.ds(img * G, G)], labv)

    gidx_v[...] = kidx_v[...] + img * NROI
    pltpu.async_copy(roi_hbm.at[gidx_v], rows_v, sem).wait()

    def col(c):
        return plsc.load_gather(rows_v, [iota, jnp.full((16,), c, jnp.int32)])

    x1, y1, x2, y2 = col(0), col(1), col(2), col(3)

    gplanes = [[gtv[c, pl.ds(0, 16)], gtv[c, pl.ds(16, 16)]] for c in range(4)]

    assigned = jnp.zeros((16,), jnp.int32)
    best = jnp.full((16,), -1.0, jnp.float32)
    area_r = (y2 - y1) * (x2 - x1)
    for g in range(G):
        xb1 = gplanes[0][g // 16][g % 16]
        yb1 = gplanes[1][g // 16][g % 16]
        xb2 = gplanes[2][g // 16][g % 16]
        yb2 = gplanes[3][g // 16][g % 16]
        ix1 = jnp.maximum(x1, xb1)
        iy1 = jnp.maximum(y1, yb1)
        ix2 = jnp.minimum(x2, xb2)
        iy2 = jnp.minimum(y2, yb2)
        inter = (iy2 - iy1) * (ix2 - ix1)
        gt_area = (yb2 - yb1) * (xb2 - xb1)
        union = area_r + gt_area - inter
        ok = (ix1 < ix2) & (iy1 < iy2)
        iou = jnp.where(ok, inter / union, 0.0)
        upd = iou > best
        assigned = jnp.where(upd, g, assigned)
        best = jnp.where(upd, iou, best)

    def gcol(c):
        return plsc.load_gather(gtv, [jnp.full((16,), c, jnp.int32), assigned])

    bx1, by1, bx2, by2 = gcol(0), gcol(1), gcol(2), gcol(3)

    slot = (wid % 8) * 16 + iota
    pos_valid = (slot < N_POS) & (kval_v[...] >= POS_TH)
    label = plsc.load_gather(labv, [assigned])
    label = jnp.where(pos_valid, label, 0)

    eps = jnp.float32(1.1920929e-07)
    w = x2 - x1
    h = y2 - y1
    cx = x1 + 0.5 * w
    cy = y1 + 0.5 * h
    bw = bx2 - bx1
    bh = by2 - by1
    bcx = bx1 + 0.5 * bw
    bcy = by1 + 0.5 * bh
    wc = jnp.maximum(w, eps)
    hc = jnp.maximum(h, eps)
    dx = (bcx - cx) / wc
    dy = (bcy - cy) / hc
    dw = _ln(bw / wc)
    dh = _ln(bh / hc)

    def scat(ref, c, v):
        plsc.store_scatter(ref, [iota, jnp.full((16,), c, jnp.int32)], v)

    scat(oroi_v, 0, x1)
    scat(oroi_v, 1, y1)
    scat(oroi_v, 2, x2)
    scat(oroi_v, 3, y2)
    scat(obox_v, 0, dx)
    scat(obox_v, 1, dy)
    scat(obox_v, 2, dw)
    scat(obox_v, 3, dh)
    olab_v[...] = label

    pltpu.sync_copy(oroi_v, oroi_hbm.at[pl.ds(base, 16)])
    pltpu.sync_copy(olab_v, olab_hbm.at[pl.ds(base, 16)])
    pltpu.sync_copy(obox_v, obox_hbm.at[pl.ds(base, 16)])


@functools.cache
def _sc_gather():
  return pl.kernel(
    _sc_body,
    out_type=[
        jax.ShapeDtypeStruct((B * N_SAMPLE, 4), jnp.float32),
        jax.ShapeDtypeStruct((B * N_SAMPLE,), jnp.int32),
        jax.ShapeDtypeStruct((B * N_SAMPLE, 4), jnp.float32),
    ],
    mesh=plsc.VectorSubcoreMesh(
        core_axis_name="c", subcore_axis_name="s",
        num_cores=2, num_subcores=16),
    scratch_types=[
        pltpu.VMEM((16,), jnp.int32),
        pltpu.VMEM((16,), jnp.float32),
        pltpu.VMEM((16,), jnp.int32),
        pltpu.VMEM((16, 16), jnp.float32),
        pltpu.VMEM((4, G), jnp.float32),
        pltpu.VMEM((G,), jnp.int32),
        pltpu.VMEM((16, 4), jnp.float32),
        pltpu.VMEM((16,), jnp.int32),
        pltpu.VMEM((16, 4), jnp.float32),
        pltpu.SemaphoreType.DMA,
    ],
    compiler_params=pltpu.CompilerParams(
        needs_layout_passes=False, use_tc_tiling_on_sc=False),
  )


def kernel(roi, gt_boxes, labels, image):
    del image
    coords = jnp.transpose(roi, (0, 2, 1))
    coords = jnp.pad(coords, ((0, 0), (0, 0), (0, NPAD - NROI)))
    coords = coords.reshape(B, 4, ROWS, LANES)
    keep_idx, keep_val = _select(coords, gt_boxes)

    roi_pad = jnp.pad(roi.reshape(B * NROI, 4), ((0, 0), (0, 12)))
    gt_planes = jnp.transpose(gt_boxes, (0, 2, 1)).reshape(B * 4, G)
    lab_flat = labels.reshape(B * G).astype(jnp.int32)
    sroi, slab, sbox = _sc_gather()(
        roi_pad, gt_planes, lab_flat,
        keep_idx.reshape(B * N_SAMPLE), keep_val.reshape(B * N_SAMPLE))
    return (sroi.reshape(B, N_SAMPLE, 4),
            slab.reshape(B, N_SAMPLE),
            sbox.reshape(B, N_SAMPLE, 4))
